# baseline (speedup 1.0000x reference)
import functools

import jax
import jax.numpy as jnp
import numpy as np
from jax.experimental import pallas as pl
from jax.experimental.pallas import tpu as pltpu

CTX = 32
DM = 1024
CB = 64
HEADS = 8
HD = DM // HEADS
N = 256
CIN = 256

_HI = jax.lax.Precision.HIGHEST


def _dot(a, b, precision=_HI):
    return jnp.dot(a, b, preferred_element_type=jnp.float32, precision=precision)


def _dot_t(a, b, precision=_HI):
    return jax.lax.dot_general(
        a, b, (((1,), (1,)), ((), ())),
        preferred_element_type=jnp.float32, precision=precision)


def _head_kernel(x_ref, cbW_ref, cbb_ref, cb_ref,
                 e1W_ref, e1b_ref, e2W_ref, e2b_ref,
                 k1W_ref, k1b_ref, k2W_ref, k2b_ref,
                 h2_ref, h2k_ref, idx_ref, lsum_ref):
    x = x_ref[...]
    cb = cb_ref[...]
    z = _dot(x, cbW_ref[...]) + cbb_ref[...]
    zn = jnp.sum(z * z, axis=1, keepdims=True)
    cbn = jnp.sum(cb * cb, axis=1)[None, :]
    cross = _dot_t(z, cb)
    dist = zn + cbn - 2.0 * cross
    mind = jnp.min(dist, axis=1, keepdims=True)
    lane = jax.lax.broadcasted_iota(jnp.int32, dist.shape, 1)
    idx = jnp.min(jnp.where(dist <= mind, lane, CB), axis=1)
    idx_ref[...] = idx[:, None]
    lsum_ref[...] = jnp.sum(mind, keepdims=True)
    h1 = jax.nn.relu(_dot(x, e1W_ref[...]) + e1b_ref[...])
    h2_ref[...] = jax.nn.relu(_dot(h1, e2W_ref[...]) + e2b_ref[...])
    h1k = jax.nn.relu(_dot(cb, k1W_ref[...]) + k1b_ref[...])
    h2k_ref[...] = jax.nn.relu(_dot(h1k, k2W_ref[...]) + k2b_ref[...])


def _layernorm(h, w, b):
    m = jnp.mean(h, axis=1, keepdims=True)
    v = jnp.mean((h - m) ** 2, axis=1, keepdims=True)
    return (h - m) / jnp.sqrt(v + 1e-5) * w + b


def _proj_kernel(h2_ref, h2k_ref, e3W_ref, e3b_ref, k3W_ref, k3b_ref,
                 elnw_ref, elnb_ref, klnw_ref, klnb_ref,
                 Wq_ref, bq_ref, Wk_ref, bk_ref, Wv_ref, bv_ref,
                 Qt_ref, Kt_ref, Vt_ref):
    h3 = _dot(h2_ref[...], e3W_ref[...]) + e3b_ref[0]
    q = _layernorm(h3, elnw_ref[...], elnb_ref[...])
    Qt_ref[0] = _dot(q, Wq_ref[...]) + bq_ref[...]
    hk3 = _dot(h2k_ref[...], k3W_ref[...]) + k3b_ref[0]
    kv = _layernorm(hk3, klnw_ref[...], klnb_ref[...])
    Kt_ref[0] = _dot(kv, Wk_ref[...]) + bk_ref[...]
    Vt_ref[0] = _dot(kv, Wv_ref[...]) + bv_ref[...]


def _attn_kernel(idx_sref, qt_ref, kt_ref, vt_ref, out_ref):
    del idx_sref
    q = qt_ref[:, 0, 0, :]
    k = kt_ref[:, 0, 0, :]
    v = vt_ref[:, 0, 0, :]
    scale = np.float32(1.0 / float(np.sqrt(HD)))
    for h in range(HEADS):
        sl = slice(h * HD, (h + 1) * HD)
        s = _dot_t(q[:, sl], k[:, sl]) * scale
        m = jnp.max(s, axis=1, keepdims=True)
        e = jnp.exp(s - m)
        w = e / jnp.sum(e, axis=1, keepdims=True)
        out_ref[0, :, sl] = _dot(w, v[:, sl])


def _run_head(x, p):
    out = pl.pallas_call(
        _head_kernel,
        out_shape=(
            jax.ShapeDtypeStruct((N, DM), jnp.float32),
            jax.ShapeDtypeStruct((CB, DM), jnp.float32),
            jax.ShapeDtypeStruct((N, 1), jnp.int32),
            jax.ShapeDtypeStruct((1, 1), jnp.float32),
        ),
    )(x, p["cb_fc_W"], p["cb_fc_b"][None, :], p["codebook"],
      p["e_p1_W"], p["e_p1_b"][None, :], p["e_p2_W"], p["e_p2_b"][None, :],
      p["k_p1_W"], p["k_p1_b"][None, :], p["k_p2_W"], p["k_p2_b"][None, :])
    return out


def _run_proj(h2, h2k, p):
    full = lambda shape: pl.BlockSpec(shape, lambda c: (0,) * len(shape))
    in_specs = [
        full((N, DM)),
        full((CB, DM)),
        pl.BlockSpec((DM, DM), lambda c: (0, c)),
        pl.BlockSpec((1, 1, DM), lambda c: (c, 0, 0)),
        pl.BlockSpec((DM, DM), lambda c: (0, c)),
        pl.BlockSpec((1, 1, DM), lambda c: (c, 0, 0)),
        full((1, DM)), full((1, DM)),
        full((1, DM)), full((1, DM)),
        full((DM, DM)), full((1, DM)),
        full((DM, DM)), full((1, DM)),
        full((DM, DM)), full((1, DM)),
    ]
    out_specs = (
        pl.BlockSpec((1, N, DM), lambda c: (c, 0, 0)),
        pl.BlockSpec((1, CB, DM), lambda c: (c, 0, 0)),
        pl.BlockSpec((1, CB, DM), lambda c: (c, 0, 0)),
    )
    return pl.pallas_call(
        _proj_kernel,
        grid=(CTX,),
        in_specs=in_specs,
        out_specs=out_specs,
        out_shape=(
            jax.ShapeDtypeStruct((CTX, N, DM), jnp.float32),
            jax.ShapeDtypeStruct((CTX, CB, DM), jnp.float32),
            jax.ShapeDtypeStruct((CTX, CB, DM), jnp.float32),
        ),
    )(h2, h2k,
      p["e_p3_W"], p["e_p3_b"].reshape(CTX, 1, DM),
      p["k_p3_W"], p["k_p3_b"].reshape(CTX, 1, DM),
      p["e_ln_w"][None, :], p["e_ln_b"][None, :],
      p["k_ln_w"][None, :], p["k_ln_b"][None, :],
      p["Wq"], p["bq"][None, :], p["Wk"], p["bk"][None, :],
      p["Wv"], p["bv"][None, :])


def _run_attn(idx, Qt, Kt, Vt):
    grid_spec = pltpu.PrefetchScalarGridSpec(
        num_scalar_prefetch=1,
        grid=(N,),
        in_specs=[
            pl.BlockSpec((CTX, 1, 1, DM), lambda f, idxr: (0, f, 0, 0)),
            pl.BlockSpec((CTX, 1, 1, DM), lambda f, idxr: (0, idxr[f], 0, 0)),
            pl.BlockSpec((CTX, 1, 1, DM), lambda f, idxr: (0, idxr[f], 0, 0)),
        ],
        out_specs=pl.BlockSpec((1, CTX, DM), lambda f, idxr: (f, 0, 0)),
    )
    return pl.pallas_call(
        _attn_kernel,
        grid_spec=grid_spec,
        out_shape=jax.ShapeDtypeStruct((N, CTX, DM), jnp.float32),
    )(idx, Qt.reshape(CTX, N, 1, DM), Kt.reshape(CTX, CB, 1, DM),
      Vt.reshape(CTX, CB, 1, DM))


def kernel(emo_prompts, params):
    b, f = emo_prompts.shape[0], emo_prompts.shape[1]
    x = emo_prompts.reshape(N, CIN)
    h2, h2k, idx2, lsum = _run_head(x, params)
    Qt, Kt, Vt = _run_proj(h2, h2k, params)
    out = _run_attn(idx2.reshape(N), Qt, Kt, Vt)
    final = out.reshape(b, f, CTX, DM)
    m = lsum[0, 0] / np.float32(N * DM)
    vq_loss = m + 0.25 * m
    return final, vq_loss


# speedup vs baseline: 6.1310x; 6.1310x over previous
import jax
import jax.numpy as jnp
import numpy as np
from jax.experimental import pallas as pl
from jax.experimental.pallas import tpu as pltpu

CTX = 32
DM = 1024
CB = 64
HEADS = 8
HD = DM // HEADS
N = 256
CIN = 256
FB = 16

_HI = jax.lax.Precision.HIGHEST


def _dot(a, b, precision=None):
    return jnp.dot(a, b, preferred_element_type=jnp.float32, precision=precision)


def _dot_t(a, b, precision=None):
    return jax.lax.dot_general(
        a, b, (((1,), (1,)), ((), ())),
        preferred_element_type=jnp.float32, precision=precision)


def _head_kernel(x_ref, cbW_ref, cbb_ref, cb_ref,
                 e1W_ref, e1b_ref, e2W_ref, e2b_ref,
                 k1W_ref, k1b_ref, k2W_ref, k2b_ref,
                 h2_ref, h2k_ref, idx_ref, lsum_ref):
    x = x_ref[...]
    cb = cb_ref[...]
    z = _dot(x, cbW_ref[...], precision=_HI) + cbb_ref[...]
    zn = jnp.sum(z * z, axis=1, keepdims=True)
    cbn = jnp.sum(cb * cb, axis=1)[None, :]
    cross = _dot_t(z, cb, precision=_HI)
    dist = zn + cbn - 2.0 * cross
    mind = jnp.min(dist, axis=1, keepdims=True)
    lane = jax.lax.broadcasted_iota(jnp.int32, dist.shape, 1)
    idx = jnp.min(jnp.where(dist <= mind, lane, CB), axis=1)
    idx_ref[...] = idx[:, None]
    lsum_ref[...] = jnp.sum(mind, keepdims=True)
    h1 = jax.nn.relu(_dot(x, e1W_ref[...]) + e1b_ref[...])
    h2_ref[...] = jax.nn.relu(_dot(h1, e2W_ref[...]) + e2b_ref[...])
    h1k = jax.nn.relu(_dot(cb, k1W_ref[...]) + k1b_ref[...])
    h2k_ref[...] = jax.nn.relu(_dot(h1k, k2W_ref[...]) + k2b_ref[...])


def _layernorm(h, w, b):
    m = jnp.mean(h, axis=1, keepdims=True)
    v = jnp.mean((h - m) ** 2, axis=1, keepdims=True)
    return (h - m) / jnp.sqrt(v + 1e-5) * w + b


def _proj_kernel(h2_ref, h2k_ref, e3W_ref, e3b_ref, k3W_ref, k3b_ref,
                 elnw_ref, elnb_ref, klnw_ref, klnb_ref,
                 Wq_ref, bq_ref, Wk_ref, bk_ref, Wv_ref, bv_ref,
                 Qs_ref, Ks_ref, Vs_ref):
    h3 = _dot(h2_ref[...], e3W_ref[...]) + e3b_ref[0]
    q = _layernorm(h3, elnw_ref[...], elnb_ref[...])
    Q = _dot(q, Wq_ref[...]) + bq_ref[...]
    hk3 = _dot(h2k_ref[...], k3W_ref[...]) + k3b_ref[0]
    kv = _layernorm(hk3, klnw_ref[...], klnb_ref[...])
    K = _dot(kv, Wk_ref[...]) + bk_ref[...]
    V = _dot(kv, Wv_ref[...]) + bv_ref[...]
    for h in range(HEADS):
        sl = slice(h * HD, (h + 1) * HD)
        Qs_ref[h, 0] = Q[:, sl]
        Ks_ref[:, h, 0, 0, :] = K[:, sl]
        Vs_ref[:, h, 0, 0, :] = V[:, sl]


def _attn_kernel(idx_sref, qs_ref, ks_ref, vs_ref, out_ref):
    scale = np.float32(1.0 / float(np.sqrt(HD)))
    neg = np.float32(-1e30)
    S = HEADS * CTX
    rh = jax.lax.broadcasted_iota(jnp.int32, (S, S), 0) // CTX
    ch = jax.lax.broadcasted_iota(jnp.int32, (S, S), 1) // CTX
    same_head = rh == ch
    fb = pl.program_id(0)
    for j in range(FB):
        e = idx_sref[fb * FB + j]
        qst = qs_ref[:, :, j, :].reshape(S, HD)
        kst = ks_ref[e].reshape(S, HD)
        vst = vs_ref[e].reshape(S, HD)
        s = _dot_t(qst, kst) * scale
        s = jnp.where(same_head, s, neg)
        m = jnp.max(s, axis=1, keepdims=True)
        p = jnp.exp(s - m)
        w = p / jnp.sum(p, axis=1, keepdims=True)
        o = _dot(w, vst)
        for h in range(HEADS):
            out_ref[j, :, h * HD:(h + 1) * HD] = o[h * CTX:(h + 1) * CTX, :]


def _run_head(x, p):
    return pl.pallas_call(
        _head_kernel,
        out_shape=(
            jax.ShapeDtypeStruct((N, DM), jnp.float32),
            jax.ShapeDtypeStruct((CB, DM), jnp.float32),
            jax.ShapeDtypeStruct((N, 1), jnp.int32),
            jax.ShapeDtypeStruct((1, 1), jnp.float32),
        ),
    )(x, p["cb_fc_W"], p["cb_fc_b"][None, :], p["codebook"],
      p["e_p1_W"], p["e_p1_b"][None, :], p["e_p2_W"], p["e_p2_b"][None, :],
      p["k_p1_W"], p["k_p1_b"][None, :], p["k_p2_W"], p["k_p2_b"][None, :])


def _run_proj(h2, h2k, p):
    full = lambda shape: pl.BlockSpec(shape, lambda c: (0,) * len(shape))
    in_specs = [
        full((N, DM)),
        full((CB, DM)),
        pl.BlockSpec((DM, DM), lambda c: (0, c)),
        pl.BlockSpec((1, 1, DM), lambda c: (c, 0, 0)),
        pl.BlockSpec((DM, DM), lambda c: (0, c)),
        pl.BlockSpec((1, 1, DM), lambda c: (c, 0, 0)),
        full((1, DM)), full((1, DM)),
        full((1, DM)), full((1, DM)),
        full((DM, DM)), full((1, DM)),
        full((DM, DM)), full((1, DM)),
        full((DM, DM)), full((1, DM)),
    ]
    out_specs = (
        pl.BlockSpec((HEADS, 1, N, HD), lambda c: (0, c, 0, 0)),
        pl.BlockSpec((CB, HEADS, 1, 1, HD), lambda c: (0, 0, c, 0, 0)),
        pl.BlockSpec((CB, HEADS, 1, 1, HD), lambda c: (0, 0, c, 0, 0)),
    )
    return pl.pallas_call(
        _proj_kernel,
        grid=(CTX,),
        in_specs=in_specs,
        out_specs=out_specs,
        out_shape=(
            jax.ShapeDtypeStruct((HEADS, CTX, N, HD), jnp.float32),
            jax.ShapeDtypeStruct((CB, HEADS, CTX, 1, HD), jnp.float32),
            jax.ShapeDtypeStruct((CB, HEADS, CTX, 1, HD), jnp.float32),
        ),
    )(h2, h2k,
      p["e_p3_W"], p["e_p3_b"].reshape(CTX, 1, DM),
      p["k_p3_W"], p["k_p3_b"].reshape(CTX, 1, DM),
      p["e_ln_w"][None, :], p["e_ln_b"][None, :],
      p["k_ln_w"][None, :], p["k_ln_b"][None, :],
      p["Wq"], p["bq"][None, :], p["Wk"], p["bk"][None, :],
      p["Wv"], p["bv"][None, :])


def _run_attn(idx, Qs, Ks, Vs):
    grid_spec = pltpu.PrefetchScalarGridSpec(
        num_scalar_prefetch=1,
        grid=(N // FB,),
        in_specs=[
            pl.BlockSpec((HEADS, CTX, FB, HD), lambda fb, idxr: (0, 0, fb, 0)),
            pl.BlockSpec((CB, HEADS, CTX, 1, HD), lambda fb, idxr: (0, 0, 0, 0, 0)),
            pl.BlockSpec((CB, HEADS, CTX, 1, HD), lambda fb, idxr: (0, 0, 0, 0, 0)),
        ],
        out_specs=pl.BlockSpec((FB, CTX, DM), lambda fb, idxr: (fb, 0, 0)),
    )
    return pl.pallas_call(
        _attn_kernel,
        grid_spec=grid_spec,
        out_shape=jax.ShapeDtypeStruct((N, CTX, DM), jnp.float32),
    )(idx, Qs, Ks, Vs)


def kernel(emo_prompts, params):
    b, f = emo_prompts.shape[0], emo_prompts.shape[1]
    x = emo_prompts.reshape(N, CIN)
    h2, h2k, idx2, lsum = _run_head(x, params)
    Qs, Ks, Vs = _run_proj(h2, h2k, params)
    out = _run_attn(idx2.reshape(N), Qs, Ks, Vs)
    final = out.reshape(b, f, CTX, DM)
    m = lsum[0, 0] / np.float32(N * DM)
    vq_loss = m + 0.25 * m
    return final, vq_loss
